# Initial kernel scaffold; baseline (speedup 1.0000x reference)
import jax
import jax.numpy as jnp
from jax.experimental import pallas as pl


def kernel(logits, attn_scores, gen_tokens):
    raise NotImplementedError("write your pallas kernel here")


# speedup vs baseline: 32.4609x; 32.4609x over previous
import functools
import math

import jax
import jax.numpy as jnp
from jax.experimental import pallas as pl
from jax.experimental.pallas import tpu as pltpu

LN2 = math.log(2.0)
PADV = -1e9
NPAD = 100096
CTEMP, CTOPP, CTOPK, CMINP = 0.666, 0.9, 27.0, 0.03
ATL, ATA, ATG = 0.3, 0.2, 0.2
ATP, ATKI, ATKA, AMP = 0.1, 0.3, 0.2, 0.5


def _attn_kernel(attn_ref, out_ref):
    l = pl.program_id(0)
    x = attn_ref[...]
    m = jnp.max(x, axis=-1, keepdims=True)
    e = jnp.exp(x - m)
    z = jnp.sum(e, axis=-1, keepdims=True)
    ap = e / z
    ent = -jnp.sum(ap * (jnp.log(jnp.clip(ap, 1e-10, 1.0)) * (1.0 / LN2)),
                   axis=-1, keepdims=True)
    ent_sum = jnp.sum(ent)
    ent_bh = jnp.reshape(ent, (16, 16))
    mh = jnp.mean(ent_bh, axis=1, keepdims=True)
    vent_sum = jnp.sum((ent_bh - mh) ** 2) / 15.0
    ap3 = jnp.reshape(ap, (16, 16, 2048))
    mu = jnp.mean(ap3, axis=0, keepdims=True)
    agr_sum = jnp.sum(jnp.abs(ap3 - mu))
    int_sum = jnp.sum(jnp.abs(x))
    lane = jax.lax.broadcasted_iota(jnp.int32, (1, 128), 1)
    vec = (jnp.where(lane == 0, ent_sum, 0.0)
           + jnp.where(lane == 1, vent_sum, 0.0)
           + jnp.where(lane == 2, agr_sum, 0.0)
           + jnp.where(lane == 3, int_sum, 0.0))

    @pl.when(l == 0)
    def _():
        out_ref[...] = vec

    @pl.when(l > 0)
    def _():
        out_ref[...] += vec


def _main_kernel(x_ref, met_ref, out_ref, samp_ref, xw_ref, vals_ref):
    x = x_ref[...]
    lane = jax.lax.broadcasted_iota(jnp.int32, (1, 128), 1)
    met = met_ref[...]
    attn_ent = jnp.sum(jnp.where(lane == 0, met, 0.0)) / 2048.0
    attn_vent = jnp.sum(jnp.where(lane == 1, met, 0.0)) / 128.0
    agreement = jnp.sum(jnp.where(lane == 2, met, 0.0)) / 4194304.0
    interaction = jnp.sum(jnp.where(lane == 3, met, 0.0)) / 4194304.0

    m = jnp.max(x, axis=-1, keepdims=True)
    u = x - m
    e = jnp.exp(u)
    z = jnp.sum(e, axis=-1, keepdims=True)
    eu = e * u
    a = jnp.sum(eu, axis=-1, keepdims=True) / z
    b = jnp.sum(eu * u, axis=-1, keepdims=True) / z
    logz = jnp.log(z)
    ent_rows = (logz - a) * (1.0 / LN2)
    vent_rows = (b - a * a) * (1.0 / (LN2 * LN2))
    lu = jnp.mean(ent_rows) + jnp.mean(vent_rows)
    au = attn_ent + attn_vent

    temperature = CTEMP * (1.0 + ATL * lu + ATA * au - ATG * agreement)
    top_p = jnp.clip(CTOPP * (1.0 + ATP * attn_vent), 0.1, 1.0)
    min_p = jnp.clip(CMINP * (1.0 - AMP * lu), 0.01, 0.5)
    top_k = jnp.clip(jnp.round(CTOPK * (1.0 + ATKI * interaction
                                        - ATKA * agreement)), 1.0, 100.0
                     ).astype(jnp.int32)

    col = jax.lax.broadcasted_iota(jnp.int32, (16, NPAD), 1)
    idx = jnp.min(jnp.where(x == m, col, NPAD), axis=-1, keepdims=True)
    samp_ref[...] = jnp.broadcast_to(idx, (16, 128))

    xw_ref[...] = x
    vals_ref[...] = jnp.zeros((16, 128), jnp.float32)

    def body(j, _):
        xx = xw_ref[...]
        mj = jnp.max(xx, axis=-1, keepdims=True)
        idxj = jnp.min(jnp.where(xx == mj, col, NPAD), axis=-1, keepdims=True)
        vals_ref[...] += jnp.where(lane == j, mj, 0.0)
        xw_ref[...] = jnp.where(col == idxj, PADV, xx)
        return 0

    jax.lax.fori_loop(0, 100, body, 0)

    sv = vals_ref[...] / temperature
    srm = m / temperature
    valid = lane < 100
    thresh = jnp.sum(jnp.where(lane == top_k - 1, sv, 0.0),
                     axis=-1, keepdims=True)
    kept1 = valid & (sv >= thresh)
    ev = jnp.where(kept1, jnp.exp(sv - srm), 0.0)
    ii = jax.lax.broadcasted_iota(jnp.int32, (128, 128), 0)
    jj = jax.lax.broadcasted_iota(jnp.int32, (128, 128), 1)
    tri = (ii <= jj).astype(jnp.float32)
    prefix = jax.lax.dot_general(ev, tri, (((1,), (0,)), ((), ())),
                                 precision=jax.lax.Precision.HIGHEST,
                                 preferred_element_type=jnp.float32)
    s1 = jnp.sum(ev, axis=-1, keepdims=True)
    keep2 = kept1 & ((prefix - ev) <= top_p * s1)
    e2 = jnp.where(keep2, ev, 0.0)
    s2 = jnp.sum(e2, axis=-1, keepdims=True)
    keep3 = keep2 & (e2 >= min_p * s2)
    e3 = jnp.where(keep3, e2, 0.0)
    s3 = jnp.sum(e3, axis=-1, keepdims=True)
    cutoff = jnp.min(jnp.where(keep3, sv, jnp.inf), axis=-1, keepdims=True)
    k3 = jnp.sum(keep3.astype(jnp.float32), axis=-1, keepdims=True)
    inv3 = 1.0 / s3

    xs = x / temperature
    gt = xs > cutoff
    eq = xs == cutoff
    n_gt = jnp.sum(gt.astype(jnp.float32), axis=-1, keepdims=True)
    n_c = jnp.round(k3 - n_gt).astype(jnp.int32)
    last = jnp.full((16, 1), -1, jnp.int32)
    idx_cut = jnp.full((16, 1), -1, jnp.int32)
    for t in range(1, 9):
        idx_t = jnp.min(jnp.where(eq & (col > last), col, NPAD),
                        axis=-1, keepdims=True)
        idx_cut = jnp.where(n_c == t, idx_t, idx_cut)
        last = idx_t
    idx_cut = jnp.where(n_c > 8, NPAD, idx_cut)
    keep = gt | (eq & (col <= idx_cut))
    out_ref[...] = jnp.where(keep, jnp.exp(xs - srm) * inv3, 0.0)


@jax.jit
def kernel(logits, attn_scores, gen_tokens):
    del gen_tokens
    current = attn_scores[:, :, :, -1, :].reshape(2048, 2048)
    met = pl.pallas_call(
        _attn_kernel,
        grid=(8,),
        in_specs=[pl.BlockSpec((256, 2048), lambda l: (l, 0))],
        out_specs=pl.BlockSpec((1, 128), lambda l: (0, 0)),
        out_shape=jax.ShapeDtypeStruct((1, 128), jnp.float32),
    )(current)

    xp = jnp.concatenate(
        [logits, jnp.full((16, NPAD - 100000), PADV, jnp.float32)], axis=1)
    probs, samp = pl.pallas_call(
        _main_kernel,
        in_specs=[pl.BlockSpec((16, NPAD), lambda: (0, 0)),
                  pl.BlockSpec((1, 128), lambda: (0, 0))],
        out_specs=[pl.BlockSpec((16, NPAD), lambda: (0, 0)),
                   pl.BlockSpec((16, 128), lambda: (0, 0))],
        out_shape=[jax.ShapeDtypeStruct((16, NPAD), jnp.float32),
                   jax.ShapeDtypeStruct((16, 128), jnp.int32)],
        scratch_shapes=[pltpu.VMEM((16, NPAD), jnp.float32),
                        pltpu.VMEM((16, 128), jnp.float32)],
    )(xp, met)
    return probs[:, :100000], samp[:, 0]
